# baseline (speedup 1.0000x reference)
import functools

import jax
import jax.numpy as jnp
from jax import lax
from jax.experimental import pallas as pl
from jax.experimental.pallas import tpu as pltpu
from jax.experimental.pallas import tpu_sc as plsc

N = 10000
C = 128
NS = 16
NK = 15
SIGMA = 0.3
SCALE = SIGMA ** 2 * 2 + 1e-10

NPAD = 10240
Q = 256
NT = NPAD // Q
BIGC = 1e4

B = NPAD * NS
NW = 32
BPW = B // NW
CH = 256
NCH = BPW // CH


DEPTH = 5
NG = NPAD // 128
BIGF = 3.0e38


def _knn_body(pq_ref, pT_ref, idx_ref, d_ref):
    q = pq_ref[...]
    pT = pT_ref[...]
    qp = lax.dot_general(q, pT, (((1,), (0,)), ((), ())),
                         precision=lax.Precision.HIGHEST,
                         preferred_element_type=jnp.float32)
    pn = jnp.sum(pT * pT, axis=0, keepdims=True)
    d_ref[...] = pn - 2.0 * qp

    m = [jnp.full((Q, 128), BIGF, jnp.float32) for _ in range(DEPTH)]
    g = [jnp.zeros((Q, 128), jnp.int32) for _ in range(DEPTH)]
    for gi in range(NG):
        t = d_ref[:, 128 * gi:128 * (gi + 1)]
        tg = jnp.full((Q, 128), gi, jnp.int32)
        for k in range(DEPTH):
            lt = t < m[k]
            m[k], t = jnp.where(lt, t, m[k]), jnp.where(lt, m[k], t)
            g[k], tg = jnp.where(lt, tg, g[k]), jnp.where(lt, g[k], tg)

    lane = lax.broadcasted_iota(jnp.int32, (Q, 128), 1)
    jc = [g[k] * 128 + lane for k in range(DEPTH)]

    kiota = lax.broadcasted_iota(jnp.int32, (Q, NS), 1)
    acc = jnp.zeros((Q, NS), jnp.int32)
    for s in range(NS):
        mm = m[0]
        for k in range(1, DEPTH):
            mm = jnp.minimum(mm, m[k])
        jbest = jnp.full((Q, 128), NPAD, jnp.int32)
        for k in range(DEPTH):
            jbest = jnp.minimum(jbest, jnp.where(m[k] == mm, jc[k], NPAD))
        mv = jnp.min(mm, axis=1, keepdims=True)
        sel = jnp.where(mm <= mv, jbest, NPAD)
        ji = jnp.min(sel, axis=1, keepdims=True)
        for k in range(DEPTH):
            m[k] = jnp.where(jc[k] == ji, BIGF, m[k])
        acc = jnp.where(kiota == s, ji, acc)
    idx_ref[...] = acc


def _knn(pq8, pT8):
    return pl.pallas_call(
        _knn_body,
        grid=(NT,),
        in_specs=[
            pl.BlockSpec((Q, 8), lambda i: (i, 0)),
            pl.BlockSpec((8, NPAD), lambda i: (0, 0)),
        ],
        out_specs=pl.BlockSpec((Q, NS), lambda i: (i, 0)),
        out_shape=jax.ShapeDtypeStruct((NPAD, NS), jnp.int32),
        scratch_shapes=[pltpu.VMEM((Q, NPAD), jnp.float32)],
    )(pq8, pT8)


@functools.partial(
    pl.kernel,
    mesh=plsc.VectorSubcoreMesh(core_axis_name="c", subcore_axis_name="s"),
    out_type=[
        jax.ShapeDtypeStruct((B, C), jnp.float32),
        jax.ShapeDtypeStruct((B, C), jnp.float32),
    ],
    scratch_types=[
        pltpu.VMEM((CH,), jnp.int32),
        pltpu.VMEM((CH, C), jnp.float32),
        pltpu.VMEM((CH, C), jnp.float32),
        pltpu.SemaphoreType.DMA,
        pltpu.SemaphoreType.DMA,
    ],
)
def _sc_gather(xt_hbm, pt_hbm, idx_hbm, outx_hbm, outp_hbm,
               idx_v, rx_v, rp_v, semx, semp):
    wid = lax.axis_index("s") * 2 + lax.axis_index("c")
    base = wid * BPW

    def body(i, carry):
        off = base + i * CH
        pltpu.sync_copy(idx_hbm.at[pl.ds(off, CH)], idx_v)
        cx = pltpu.async_copy(xt_hbm.at[idx_v], rx_v, semx)
        cp = pltpu.async_copy(pt_hbm.at[idx_v], rp_v, semp)
        cx.wait()
        cp.wait()
        pltpu.sync_copy(rx_v, outx_hbm.at[pl.ds(off, CH)])
        pltpu.sync_copy(rp_v, outp_hbm.at[pl.ds(off, CH)])
        return carry

    lax.fori_loop(0, NCH, body, 0)


def _kpconv_body(xj_ref, pjr_ref, pq_ref, kpT_ref, wdt_ref, bdw_ref,
                 y_ref, sums_ref):
    i = pl.program_id(0)
    xj = xj_ref[...]
    pjr = pjr_ref[...]
    pq = pq_ref[...]

    diff = pjr - pq[:, None, :]
    l2sq = jnp.sum(diff * diff, axis=2)
    denom = jnp.sqrt(jnp.max(l2sq, axis=1, keepdims=True)) + 1e-10
    inv = 1.0 / denom
    phn = l2sq * (inv * inv)

    ph2 = (diff * inv[:, :, None]).reshape(Q * NS, 16)
    kpT = kpT_ref[...]
    kpn = jnp.sum(kpT * kpT, axis=0, keepdims=True)
    dotk = lax.dot_general(ph2, kpT, (((1,), (0,)), ((), ())),
                           precision=lax.Precision.HIGHEST,
                           preferred_element_type=jnp.float32)
    sqr = phn.reshape(Q * NS, 1) + kpn - 2.0 * dotk
    corr = jnp.exp(-sqr / SCALE)
    A = lax.dot_general(corr, wdt_ref[...], (((1,), (0,)), ((), ())),
                        precision=lax.Precision.HIGHEST,
                        preferred_element_type=jnp.float32)
    y = jnp.sum(A.reshape(Q, NS, C) * xj, axis=1) + bdw_ref[...]
    y_ref[...] = y

    rows = i * Q + lax.broadcasted_iota(jnp.int32, (Q, 1), 0)
    ym = jnp.where(rows < N, y, 0.0)
    s1 = jnp.sum(ym, axis=0, keepdims=True)
    s2 = jnp.sum(ym * ym, axis=0, keepdims=True)
    block = jnp.concatenate([s1, s2, jnp.zeros((6, C), jnp.float32)], axis=0)

    @pl.when(i == 0)
    def _():
        sums_ref[...] = block

    @pl.when(i > 0)
    def _():
        sums_ref[...] += block


def _kpconv(xj3, pjr3, p16, kpT, wdt, bdw2):
    return pl.pallas_call(
        _kpconv_body,
        grid=(NT,),
        in_specs=[
            pl.BlockSpec((Q, NS, C), lambda i: (i, 0, 0)),
            pl.BlockSpec((Q, NS, 16), lambda i: (i, 0, 0)),
            pl.BlockSpec((Q, 16), lambda i: (i, 0)),
            pl.BlockSpec((16, C), lambda i: (0, 0)),
            pl.BlockSpec((C, C), lambda i: (0, 0)),
            pl.BlockSpec((1, C), lambda i: (0, 0)),
        ],
        out_specs=[
            pl.BlockSpec((Q, C), lambda i: (i, 0)),
            pl.BlockSpec((8, C), lambda i: (0, 0)),
        ],
        out_shape=[
            jax.ShapeDtypeStruct((NPAD, C), jnp.float32),
            jax.ShapeDtypeStruct((8, C), jnp.float32),
        ],
    )(xj3, pjr3, p16, kpT, wdt, bdw2)


def _bn_body(y_ref, sums_ref, gamma_ref, beta_ref, out_ref):
    s = sums_ref[...]
    mean = s[0:1, :] * (1.0 / N)
    var = s[1:2, :] * (1.0 / N) - mean * mean
    inv = gamma_ref[...] * lax.rsqrt(var + 1e-5)
    out_ref[...] = jnp.maximum((y_ref[...] - mean) * inv + beta_ref[...], 0.0)


def _bn(y, sums, gamma2, beta2):
    return pl.pallas_call(
        _bn_body,
        grid=(NT,),
        in_specs=[
            pl.BlockSpec((Q, C), lambda i: (i, 0)),
            pl.BlockSpec((8, C), lambda i: (0, 0)),
            pl.BlockSpec((1, C), lambda i: (0, 0)),
            pl.BlockSpec((1, C), lambda i: (0, 0)),
        ],
        out_specs=pl.BlockSpec((Q, C), lambda i: (i, 0)),
        out_shape=jax.ShapeDtypeStruct((NPAD, C), jnp.float32),
    )(y, sums, gamma2, beta2)


def kernel(p, x, o, kernel_point, W_dw, b_dw, gamma, beta):
    pq8 = jnp.full((NPAD, 8), 0.0, jnp.float32)
    pq8 = pq8.at[:N, :3].set(p)
    pq8 = pq8.at[N:, :3].set(BIGC)
    pT8 = pq8.T

    p128 = jnp.zeros((NPAD, C), jnp.float32).at[:N, :3].set(p)
    p16 = jnp.zeros((NPAD, 16), jnp.float32).at[:N, :3].set(p)
    x_pad = jnp.zeros((NPAD, C), jnp.float32).at[:N, :].set(x)

    kpT = jnp.zeros((16, C), jnp.float32).at[:3, :NK].set(kernel_point[0].T)
    wdt = jnp.zeros((C, C), jnp.float32).at[:NK, :].set(W_dw.T)
    bdw2 = b_dw.reshape(1, C)
    gamma2 = gamma.reshape(1, C)
    beta2 = beta.reshape(1, C)

    idx = _knn(pq8, pT8)
    idx_flat = idx.reshape(-1)

    gx, gp = _sc_gather(x_pad, p128, idx_flat)
    xj3 = gx.reshape(NPAD, NS, C)
    pjr3 = gp.reshape(NPAD, NS, C)[:, :, :16]

    y_pre, sums = _kpconv(xj3, pjr3, p16, kpT, wdt, bdw2)
    y = _bn(y_pre, sums, gamma2, beta2)

    return (p, y[:N], o)


# speedup vs baseline: 1.0328x; 1.0328x over previous
import functools

import jax
import jax.numpy as jnp
from jax import lax
from jax.experimental import pallas as pl
from jax.experimental.pallas import tpu as pltpu
from jax.experimental.pallas import tpu_sc as plsc

N = 10000
C = 128
NS = 16
NK = 15
SIGMA = 0.3
SCALE = SIGMA ** 2 * 2 + 1e-10

NPAD = 10240
Q = 256
NT = NPAD // Q
BIGC = 1e4

B = NPAD * NS
NW = 32
BPW = B // NW
CH = 160
NCH = BPW // CH


DEPTH = 5
NG = NPAD // 128
BIGF = 3.0e38


def _knn_body(pq_ref, pT_ref, idx_ref, d_ref):
    q = pq_ref[...]
    pT = pT_ref[...]
    qp = lax.dot_general(q, pT, (((1,), (0,)), ((), ())),
                         precision=lax.Precision.HIGHEST,
                         preferred_element_type=jnp.float32)
    pn = jnp.sum(pT * pT, axis=0, keepdims=True)
    d_ref[...] = pn - 2.0 * qp

    m = [jnp.full((Q, 128), BIGF, jnp.float32) for _ in range(DEPTH)]
    g = [jnp.zeros((Q, 128), jnp.int32) for _ in range(DEPTH)]
    for gi in range(NG):
        t = d_ref[:, 128 * gi:128 * (gi + 1)]
        tg = jnp.full((Q, 128), gi, jnp.int32)
        for k in range(DEPTH):
            lt = t < m[k]
            m[k], t = jnp.where(lt, t, m[k]), jnp.where(lt, m[k], t)
            g[k], tg = jnp.where(lt, tg, g[k]), jnp.where(lt, g[k], tg)

    lane = lax.broadcasted_iota(jnp.int32, (Q, 128), 1)
    jc = [g[k] * 128 + lane for k in range(DEPTH)]

    kiota = lax.broadcasted_iota(jnp.int32, (Q, NS), 1)
    acc = jnp.zeros((Q, NS), jnp.int32)
    for s in range(NS):
        mv = jnp.min(m[0], axis=1, keepdims=True)
        sel = jnp.where(m[0] <= mv, jc[0], NPAD)
        ji = jnp.min(sel, axis=1, keepdims=True)
        cond = jc[0] == ji
        for k in range(DEPTH - 1):
            m[k] = jnp.where(cond, m[k + 1], m[k])
            jc[k] = jnp.where(cond, jc[k + 1], jc[k])
        m[DEPTH - 1] = jnp.where(cond, BIGF, m[DEPTH - 1])
        acc = jnp.where(kiota == s, ji, acc)
    idx_ref[...] = acc


def _knn(pq8, pT8):
    return pl.pallas_call(
        _knn_body,
        grid=(NT,),
        in_specs=[
            pl.BlockSpec((Q, 8), lambda i: (i, 0)),
            pl.BlockSpec((8, NPAD), lambda i: (0, 0)),
        ],
        out_specs=pl.BlockSpec((Q, NS), lambda i: (i, 0)),
        out_shape=jax.ShapeDtypeStruct((NPAD, NS), jnp.int32),
        scratch_shapes=[pltpu.VMEM((Q, NPAD), jnp.float32)],
    )(pq8, pT8)


@functools.partial(
    pl.kernel,
    mesh=plsc.VectorSubcoreMesh(core_axis_name="c", subcore_axis_name="s"),
    out_type=[
        jax.ShapeDtypeStruct((B, C), jnp.float32),
        jax.ShapeDtypeStruct((B, C), jnp.float32),
    ],
    scratch_types=[
        pltpu.VMEM((BPW,), jnp.int32),
        pltpu.VMEM((CH, C), jnp.float32),
        pltpu.VMEM((CH, C), jnp.float32),
        pltpu.VMEM((CH, C), jnp.float32),
        pltpu.VMEM((CH, C), jnp.float32),
        pltpu.SemaphoreType.DMA,
        pltpu.SemaphoreType.DMA,
        pltpu.SemaphoreType.DMA,
        pltpu.SemaphoreType.DMA,
    ],
)
def _sc_gather(xt_hbm, pt_hbm, idx_hbm, outx_hbm, outp_hbm,
               idxall_v, rx0, rx1, rp0, rp1, sg0, sg1, sw0, sw1):
    wid = lax.axis_index("s") * 2 + lax.axis_index("c")
    base = wid * BPW
    pltpu.sync_copy(idx_hbm.at[pl.ds(base, BPW)], idxall_v)

    rx, rp = [rx0, rx1], [rp0, rp1]
    sg, sw = [sg0, sg1], [sw0, sw1]
    wr = [None, None]
    prev = None
    for c in range(NCH):
        b = c & 1
        if wr[b] is not None:
            wr[b][0].wait()
            wr[b][1].wait()
        iv = idxall_v.at[pl.ds(c * CH, CH)]
        g1 = pltpu.async_copy(xt_hbm.at[iv], rx[b], sg[b])
        g2 = pltpu.async_copy(pt_hbm.at[iv], rp[b], sg[b])
        if prev is not None:
            pb, pg1, pg2, poff = prev
            pg1.wait()
            pg2.wait()
            wr[pb] = (
                pltpu.async_copy(rx[pb], outx_hbm.at[pl.ds(poff, CH)], sw[pb]),
                pltpu.async_copy(rp[pb], outp_hbm.at[pl.ds(poff, CH)], sw[pb]),
            )
        prev = (b, g1, g2, base + c * CH)
    b, g1, g2, off = prev
    g1.wait()
    g2.wait()
    pltpu.async_copy(rx[b], outx_hbm.at[pl.ds(off, CH)], sw[b]).wait()
    pltpu.async_copy(rp[b], outp_hbm.at[pl.ds(off, CH)], sw[b]).wait()
    ob = 1 - b
    if wr[ob] is not None:
        wr[ob][0].wait()
        wr[ob][1].wait()


def _kpconv_body(xj_ref, pjr_ref, pq_ref, kpT_ref, wdt_ref, bdw_ref,
                 y_ref, sums_ref):
    i = pl.program_id(0)
    xj = xj_ref[...]
    pjr = pjr_ref[...]
    pq = pq_ref[...]

    diff = pjr - pq[:, None, :]
    l2sq = jnp.sum(diff * diff, axis=2)
    denom = jnp.sqrt(jnp.max(l2sq, axis=1, keepdims=True)) + 1e-10
    inv = 1.0 / denom
    phn = l2sq * (inv * inv)

    ph2 = (diff * inv[:, :, None]).reshape(Q * NS, 16)
    kpT = kpT_ref[...]
    kpn = jnp.sum(kpT * kpT, axis=0, keepdims=True)
    dotk = lax.dot_general(ph2, kpT, (((1,), (0,)), ((), ())),
                           precision=lax.Precision.HIGHEST,
                           preferred_element_type=jnp.float32)
    sqr = phn.reshape(Q * NS, 1) + kpn - 2.0 * dotk
    corr = jnp.exp(-sqr / SCALE)
    A = lax.dot_general(corr, wdt_ref[...], (((1,), (0,)), ((), ())),
                        precision=lax.Precision.HIGHEST,
                        preferred_element_type=jnp.float32)
    y = jnp.sum(A.reshape(Q, NS, C) * xj, axis=1) + bdw_ref[...]
    y_ref[...] = y

    rows = i * Q + lax.broadcasted_iota(jnp.int32, (Q, 1), 0)
    ym = jnp.where(rows < N, y, 0.0)
    s1 = jnp.sum(ym, axis=0, keepdims=True)
    s2 = jnp.sum(ym * ym, axis=0, keepdims=True)
    block = jnp.concatenate([s1, s2, jnp.zeros((6, C), jnp.float32)], axis=0)

    @pl.when(i == 0)
    def _():
        sums_ref[...] = block

    @pl.when(i > 0)
    def _():
        sums_ref[...] += block


def _kpconv(xj3, pjr3, p16, kpT, wdt, bdw2):
    return pl.pallas_call(
        _kpconv_body,
        grid=(NT,),
        in_specs=[
            pl.BlockSpec((Q, NS, C), lambda i: (i, 0, 0)),
            pl.BlockSpec((Q, NS, 16), lambda i: (i, 0, 0)),
            pl.BlockSpec((Q, 16), lambda i: (i, 0)),
            pl.BlockSpec((16, C), lambda i: (0, 0)),
            pl.BlockSpec((C, C), lambda i: (0, 0)),
            pl.BlockSpec((1, C), lambda i: (0, 0)),
        ],
        out_specs=[
            pl.BlockSpec((Q, C), lambda i: (i, 0)),
            pl.BlockSpec((8, C), lambda i: (0, 0)),
        ],
        out_shape=[
            jax.ShapeDtypeStruct((NPAD, C), jnp.float32),
            jax.ShapeDtypeStruct((8, C), jnp.float32),
        ],
    )(xj3, pjr3, p16, kpT, wdt, bdw2)


def _bn_body(y_ref, sums_ref, gamma_ref, beta_ref, out_ref):
    s = sums_ref[...]
    mean = s[0:1, :] * (1.0 / N)
    var = s[1:2, :] * (1.0 / N) - mean * mean
    inv = gamma_ref[...] * lax.rsqrt(var + 1e-5)
    out_ref[...] = jnp.maximum((y_ref[...] - mean) * inv + beta_ref[...], 0.0)


def _bn(y, sums, gamma2, beta2):
    return pl.pallas_call(
        _bn_body,
        grid=(NT,),
        in_specs=[
            pl.BlockSpec((Q, C), lambda i: (i, 0)),
            pl.BlockSpec((8, C), lambda i: (0, 0)),
            pl.BlockSpec((1, C), lambda i: (0, 0)),
            pl.BlockSpec((1, C), lambda i: (0, 0)),
        ],
        out_specs=pl.BlockSpec((Q, C), lambda i: (i, 0)),
        out_shape=jax.ShapeDtypeStruct((NPAD, C), jnp.float32),
    )(y, sums, gamma2, beta2)


def kernel(p, x, o, kernel_point, W_dw, b_dw, gamma, beta):
    pq8 = jnp.full((NPAD, 8), 0.0, jnp.float32)
    pq8 = pq8.at[:N, :3].set(p)
    pq8 = pq8.at[N:, :3].set(BIGC)
    pT8 = pq8.T

    p128 = jnp.zeros((NPAD, C), jnp.float32).at[:N, :3].set(p)
    p16 = jnp.zeros((NPAD, 16), jnp.float32).at[:N, :3].set(p)
    x_pad = jnp.zeros((NPAD, C), jnp.float32).at[:N, :].set(x)

    kpT = jnp.zeros((16, C), jnp.float32).at[:3, :NK].set(kernel_point[0].T)
    wdt = jnp.zeros((C, C), jnp.float32).at[:NK, :].set(W_dw.T)
    bdw2 = b_dw.reshape(1, C)
    gamma2 = gamma.reshape(1, C)
    beta2 = beta.reshape(1, C)

    idx = _knn(pq8, pT8)
    idx_flat = idx.reshape(-1)

    gx, gp = _sc_gather(x_pad, p128, idx_flat)
    xj3 = gx.reshape(NPAD, NS, C)
    pjr3 = gp.reshape(NPAD, NS, C)[:, :, :16]

    y_pre, sums = _kpconv(xj3, pjr3, p16, kpT, wdt, bdw2)
    y = _bn(y_pre, sums, gamma2, beta2)

    return (p, y[:N], o)
